# baseline (speedup 1.0000x reference)
import functools

import jax
import jax.numpy as jnp
from jax import lax
from jax.experimental import pallas as pl
from jax.experimental.pallas import tpu as pltpu
from jax.experimental.pallas import tpu_sc as plsc

L = 16
NW = 32
SLOTS = 8
BIG = 1e10
SENT = 1e5
PAD = 2 * L


def _sc_chamfer(B, N):
    mesh = plsc.VectorSubcoreMesh(core_axis_name="c", subcore_axis_name="s")
    NCH = N // L

    @functools.partial(
        pl.kernel,
        mesh=mesh,
        out_type=jax.ShapeDtypeStruct((NW * L,), jnp.float32),
        scratch_types=[
            pltpu.VMEM((N,), jnp.float32),
            pltpu.VMEM((N,), jnp.float32),
            pltpu.VMEM((N,), jnp.float32),
            pltpu.VMEM((N,), jnp.float32),
            pltpu.VMEM((N,), jnp.float32),
            pltpu.VMEM((N,), jnp.float32),
            pltpu.VMEM((N,), jnp.int32),
            pltpu.VMEM((N + PAD,), jnp.float32),
            pltpu.VMEM((N + PAD,), jnp.float32),
            pltpu.VMEM((N + PAD,), jnp.float32),
            pltpu.VMEM((N + PAD,), jnp.float32),
            pltpu.VMEM((N + PAD,), jnp.float32),
            pltpu.VMEM((N + PAD,), jnp.float32),
            pltpu.VMEM((L,), jnp.float32),
        ],
        compiler_params=pltpu.CompilerParams(needs_layout_passes=False),
    )
    def cham(z_hbm, ax_hbm, ay_hbm, gx_hbm, gy_hbm, gz_hbm, m_hbm, out_hbm,
             z_v, ax_v, ay_v, gx_v, gy_v, gz_v, m_v,
             cpx, cpy, cpz, cgx, cgy, cgz, acc_v):
        cid = lax.axis_index("c")
        sid = lax.axis_index("s")
        wid = sid * 2 + cid
        bat = wid // SLOTS
        slot = wid % SLOTS
        boff = bat * N

        pltpu.sync_copy(z_hbm.at[pl.ds(boff, N)], z_v)
        pltpu.sync_copy(ax_hbm, ax_v)
        pltpu.sync_copy(ay_hbm, ay_v)
        pltpu.sync_copy(gx_hbm.at[pl.ds(boff, N)], gx_v)
        pltpu.sync_copy(gy_hbm.at[pl.ds(boff, N)], gy_v)
        pltpu.sync_copy(gz_hbm.at[pl.ds(boff, N)], gz_v)
        pltpu.sync_copy(m_hbm.at[pl.ds(boff, N)], m_v)

        def comp_body(i, carry):
            n_p, n_g = carry
            sl = pl.ds(i * L, L)
            zc = z_v[sl]
            pxc = ax_v[sl] * zc
            pyc = ay_v[sl] * zc
            gxc = gx_v[sl]
            gyc = gy_v[sl]
            gzc = gz_v[sl]
            mc = m_v[sl] > 0
            mp = mc & (pxc + pyc + zc != 0.0)
            mg = mc & (gxc + gyc + gzc != 0.0)
            mpi = mp.astype(jnp.int32)
            mgi = mg.astype(jnp.int32)
            pidx = n_p + (plsc.cumsum(mpi) - mpi)
            gidx = n_g + (plsc.cumsum(mgi) - mgi)
            plsc.store_scatter(cpx, [pidx], pxc, mask=mp)
            plsc.store_scatter(cpy, [pidx], pyc, mask=mp)
            plsc.store_scatter(cpz, [pidx], zc, mask=mp)
            plsc.store_scatter(cgx, [gidx], gxc, mask=mg)
            plsc.store_scatter(cgy, [gidx], gyc, mask=mg)
            plsc.store_scatter(cgz, [gidx], gzc, mask=mg)
            return (n_p + jnp.sum(mpi), n_g + jnp.sum(mgi))

        n_p, n_g = lax.fori_loop(0, NCH, comp_body,
                                 (jnp.int32(0), jnp.int32(0)))

        sent = jnp.full((L,), SENT, jnp.float32)
        cpx[pl.ds(n_p, L)] = sent
        cpy[pl.ds(n_p, L)] = sent
        cpz[pl.ds(n_p, L)] = sent
        cgx[pl.ds(n_g, L)] = sent
        cgy[pl.ds(n_g, L)] = sent
        cgz[pl.ds(n_g, L)] = sent

        lane = lax.iota(jnp.int32, L)

        def direction(qx_r, qy_r, qz_r, nq, rx_r, ry_r, rz_r, nr, acc):
            qper = (nq + SLOTS - 1) // SLOTS
            qlo = slot * qper
            qhi = jnp.minimum(nq, qlo + qper)
            nblk = (jnp.maximum(0, qhi - qlo) + L - 1) // L
            nr4 = (nr + 3) // 4

            def qblock(ib, acc):
                base = qlo + ib * L
                qx = qx_r[pl.ds(base, L)]
                qy = qy_r[pl.ds(base, L)]
                qz = qz_r[pl.ds(base, L)]

                def rloop(j, rmin):
                    j4 = j * 4
                    for u in range(4):
                        jv = jnp.full((L,), j4 + u, jnp.int32)
                        rx = plsc.load_gather(rx_r, [jv])
                        ry = plsc.load_gather(ry_r, [jv])
                        rz = plsc.load_gather(rz_r, [jv])
                        dx = qx - rx
                        dy = qy - ry
                        dz = qz - rz
                        d = dx * dx + dy * dy + dz * dz
                        rmin = jnp.minimum(rmin, d)
                    return rmin

                rmin = lax.fori_loop(0, nr4, rloop,
                                     jnp.full((L,), BIG, jnp.float32))
                valid = (base + lane) < qhi
                return acc + jnp.where(valid, rmin, 0.0)

            return lax.fori_loop(0, nblk, qblock, acc)

        acc = jnp.zeros((L,), jnp.float32)
        acc = direction(cpx, cpy, cpz, n_p, cgx, cgy, cgz, n_g, acc)
        acc = direction(cgx, cgy, cgz, n_g, cpx, cpy, cpz, n_p, acc)

        acc_v[...] = acc
        pltpu.sync_copy(acc_v, out_hbm.at[pl.ds(wid * L, L)])

    return cham


def kernel(pred, gt_xyz, mask, fx, fy, cx, cy):
    B, _, H, W = pred.shape
    N = H * W
    fx = jnp.asarray(fx, jnp.float32)
    fy = jnp.asarray(fy, jnp.float32)
    cx = jnp.asarray(cx, jnp.float32)
    cy = jnp.asarray(cy, jnp.float32)

    z = pred.reshape(B * N).astype(jnp.float32)
    gx = gt_xyz[:, 0, :, :].reshape(B * N).astype(jnp.float32)
    gy = gt_xyz[:, 1, :, :].reshape(B * N).astype(jnp.float32)
    gz = gt_xyz[:, 2, :, :].reshape(B * N).astype(jnp.float32)
    m = mask.reshape(B * N).astype(jnp.int32)
    n = jnp.arange(N, dtype=jnp.int32)
    ax = ((n % W).astype(jnp.float32) - cx) / fx
    ay = ((n // W).astype(jnp.float32) - cy) / fy

    out = _sc_chamfer(B, N)(z, ax, ay, gx, gy, gz, m)
    return jnp.sum(out) / jnp.float32(B)
